# Initial kernel scaffold; baseline (speedup 1.0000x reference)
import jax
import jax.numpy as jnp
from jax.experimental import pallas as pl


def kernel(indices, C_L, Z_init_II, tok_idx, W_l, W_m, g_z, W_z, W_mlp1, W_mlp2, W_mlp3):
    raise NotImplementedError("write your pallas kernel here")


# speedup vs baseline: 7.9829x; 7.9829x over previous
import functools

import jax
import jax.numpy as jnp
from jax import lax
from jax.experimental import pallas as pl
from jax.experimental.pallas import tpu as pltpu
from jax.experimental.pallas import tpu_sc as plsc

L = 2048
K = 64
I = 512
C = 128
R = L * K

NW = 32
RPW = R // NW
CHUNK = 128
NCH = RPW // CHUNK

LB = 8
BR = LB * K


def _proj_body(c_ref, wl_ref, wm_ref, q_ref, m_ref):
    r = jnp.maximum(c_ref[...], 0.0)
    q_ref[...] = jnp.dot(r, wl_ref[...], preferred_element_type=jnp.float32)
    m_ref[...] = jnp.dot(r, wm_ref[...], preferred_element_type=jnp.float32)


def _proj(c_l, w_l, w_m):
    return pl.pallas_call(
        _proj_body,
        out_shape=[jax.ShapeDtypeStruct((L, C), jnp.float32),
                   jax.ShapeDtypeStruct((L, C), jnp.float32)],
    )(c_l, w_l, w_m)


def _gather_body(m_hbm, zf_hbm, idx_hbm, tok_hbm, gm_hbm, gz_hbm,
                 tok_v, idx_v, gidx_v, zidx_v,
                 bm0, bm1, bz0, bz1, sm0, sm1, sz0, sz1):
    cid = lax.axis_index("c")
    sid = lax.axis_index("s")
    wid = sid * 2 + cid
    base = wid * RPW

    pltpu.sync_copy(tok_hbm, tok_v)
    pltpu.sync_copy(idx_hbm.at[pl.ds(base, RPW)], idx_v)

    def prep(i, carry):
        off = i * 16
        g = idx_v[pl.ds(off, 16)]
        g = jnp.minimum(jnp.maximum(g, 0), L - 1)
        r = base + off + lax.iota(jnp.int32, 16)
        l = jax.lax.shift_right_logical(r, 6)
        tq = plsc.load_gather(tok_v, [l])
        tk = plsc.load_gather(tok_v, [g])
        gidx_v[pl.ds(off, 16)] = g
        zidx_v[pl.ds(off, 16)] = tq * I + tk
        return carry

    lax.fori_loop(0, RPW // 16, prep, 0)

    bufs = ((bm0, bz0, sm0, sz0), (bm1, bz1, sm1, sz1))

    def gather2(i, carry):
        handles = []
        for b, (bm, bz, sm, sz) in enumerate(bufs):
            cc = i * 2 + b
            hm = pltpu.async_copy(
                m_hbm.at[gidx_v.at[pl.ds(cc * CHUNK, CHUNK)]], bm, sm)
            hz = pltpu.async_copy(
                zf_hbm.at[zidx_v.at[pl.ds(cc * CHUNK, CHUNK)]], bz, sz)
            handles.append((cc, bm, bz, hm, hz))
        for cc, bm, bz, hm, hz in handles:
            out_off = base + cc * CHUNK
            hm.wait()
            pltpu.sync_copy(bm, gm_hbm.at[pl.ds(out_off, CHUNK)])
            hz.wait()
            pltpu.sync_copy(bz, gz_hbm.at[pl.ds(out_off, CHUNK)])
        return carry

    lax.fori_loop(0, NCH // 2, gather2, 0)


_gather = functools.partial(
    pl.kernel,
    mesh=plsc.VectorSubcoreMesh(core_axis_name="c", subcore_axis_name="s"),
    compiler_params=pltpu.CompilerParams(needs_layout_passes=False),
    out_type=[jax.ShapeDtypeStruct((R, C), jnp.float32),
              jax.ShapeDtypeStruct((R, C), jnp.float32)],
    scratch_types=[
        pltpu.VMEM((L,), jnp.int32),
        pltpu.VMEM((RPW,), jnp.int32),
        pltpu.VMEM((RPW,), jnp.int32),
        pltpu.VMEM((RPW,), jnp.int32),
        pltpu.VMEM((CHUNK, C), jnp.float32),
        pltpu.VMEM((CHUNK, C), jnp.float32),
        pltpu.VMEM((CHUNK, C), jnp.float32),
        pltpu.VMEM((CHUNK, C), jnp.float32),
        pltpu.SemaphoreType.DMA,
        pltpu.SemaphoreType.DMA,
        pltpu.SemaphoreType.DMA,
        pltpu.SemaphoreType.DMA,
    ],
)(_gather_body)


def _mlp_body(q_ref, gm_ref, gz_ref, gzw_ref, wz_ref, w1_ref, w2_ref, w3_ref,
              o_ref):
    q = q_ref[...]
    qb = jnp.broadcast_to(q[:, None, :], (LB, K, C)).reshape(BR, C)
    z = gz_ref[...]
    ms = jnp.mean(z * z, axis=-1, keepdims=True)
    zn = z * lax.rsqrt(ms + 1e-6) * gzw_ref[...]
    p = qb + gm_ref[...] + jnp.dot(zn, wz_ref[...],
                                   preferred_element_type=jnp.float32)
    h = jnp.dot(jnp.maximum(p, 0.0), w1_ref[...],
                preferred_element_type=jnp.float32)
    h = jnp.dot(jnp.maximum(h, 0.0), w2_ref[...],
                preferred_element_type=jnp.float32)
    h = jnp.dot(jnp.maximum(h, 0.0), w3_ref[...],
                preferred_element_type=jnp.float32)
    o_ref[...] = p + h


def _mlp(q, gm, gz, gzw, wz, w1, w2, w3):
    wspec = pl.BlockSpec((C, C), lambda i: (0, 0))
    return pl.pallas_call(
        _mlp_body,
        grid=(R // BR,),
        in_specs=[
            pl.BlockSpec((LB, C), lambda i: (i, 0)),
            pl.BlockSpec((BR, C), lambda i: (i, 0)),
            pl.BlockSpec((BR, C), lambda i: (i, 0)),
            pl.BlockSpec((1, C), lambda i: (0, 0)),
            wspec, wspec, wspec, wspec,
        ],
        out_specs=pl.BlockSpec((BR, C), lambda i: (i, 0)),
        out_shape=jax.ShapeDtypeStruct((R, C), jnp.float32),
    )(q, gm, gz, gzw, wz, w1, w2, w3)


def kernel(indices, C_L, Z_init_II, tok_idx, W_l, W_m, g_z, W_z,
           W_mlp1, W_mlp2, W_mlp3):
    idx_flat = indices.reshape(R).astype(jnp.int32)
    tok = tok_idx.astype(jnp.int32)
    zf = Z_init_II.reshape(I * I, C)
    q, m = _proj(C_L[0], W_l, W_m)
    gm, gz = _gather(m, zf, idx_flat, tok)
    out = _mlp(q, gm, gz, g_z.reshape(1, C), W_z, W_mlp1, W_mlp2, W_mlp3)
    return out.reshape(1, L, K, C)
